# baseline (speedup 1.0000x reference)
import functools

import jax
import jax.numpy as jnp
from jax import lax
from jax.experimental import pallas as pl
from jax.experimental.pallas import tpu as pltpu
from jax.experimental.pallas import tpu_sc as plsc

LANES = 16
IDX_GRP = 128
GATHERS_PER_CHUNK = 5


def _embed_sum(idx3, table, batch, ctx, embed, nw, nc):
    rows_per_w = batch // nw
    groups = rows_per_w * ctx // IDX_GRP
    chunks = groups // GATHERS_PER_CHUNK
    rows_per_chunk = GATHERS_PER_CHUNK * IDX_GRP // ctx
    mesh = plsc.VectorSubcoreMesh(core_axis_name="c", subcore_axis_name="s")

    @functools.partial(
        pl.kernel,
        out_type=jax.ShapeDtypeStruct((batch, embed), jnp.float32),
        mesh=mesh,
        scratch_types=[
            pltpu.VMEM((groups, IDX_GRP), jnp.int32),
            pltpu.VMEM((GATHERS_PER_CHUNK * IDX_GRP, embed), jnp.float32),
            pltpu.VMEM((rows_per_w, embed), jnp.float32),
            pltpu.SemaphoreType.DMA,
        ],
        compiler_params=pltpu.CompilerParams(use_tc_tiling_on_sc=False),
    )
    def body(idx_hbm, table_hbm, out_hbm, idx_v, rows_v, emb_v, sem):
        wid = lax.axis_index("s") * nc + lax.axis_index("c")
        pltpu.sync_copy(idx_hbm.at[wid], idx_v)
        for t in range(chunks):
            handles = [
                pltpu.async_copy(
                    table_hbm.at[idx_v.at[t * GATHERS_PER_CHUNK + g]],
                    rows_v.at[pl.ds(g * IDX_GRP, IDX_GRP)],
                    sem,
                )
                for g in range(GATHERS_PER_CHUNK)
            ]
            for h in handles:
                h.wait()

            def reduce_row(rr, _, t=t):
                for j in range(embed // LANES):
                    acc = rows_v[rr * ctx, pl.ds(j * LANES, LANES)]
                    for c in range(1, ctx):
                        acc = acc + rows_v[rr * ctx + c, pl.ds(j * LANES, LANES)]
                    emb_v[t * rows_per_chunk + rr, pl.ds(j * LANES, LANES)] = acc
                return 0

            lax.fori_loop(0, rows_per_chunk, reduce_row, 0)
        pltpu.sync_copy(emb_v, out_hbm.at[pl.ds(wid * rows_per_w, rows_per_w)])

    return body(idx3, table)


def _mm_body(x_ref, w_ref, o_ref):
    o_ref[...] = jnp.dot(x_ref[...], w_ref[...],
                         preferred_element_type=jnp.float32)


def _project(x, w, vt=1024):
    batch, embed = x.shape
    _, vocab = w.shape
    nv = pl.cdiv(vocab, vt)
    return pl.pallas_call(
        _mm_body,
        grid=(nv,),
        in_specs=[
            pl.BlockSpec((batch, embed), lambda i: (0, 0)),
            pl.BlockSpec((embed, vt), lambda i: (0, i)),
        ],
        out_specs=pl.BlockSpec((batch, vt), lambda i: (0, i)),
        out_shape=jax.ShapeDtypeStruct((batch, vocab), jnp.float32),
        compiler_params=pltpu.CompilerParams(
            dimension_semantics=("arbitrary",)),
    )(x, w)


def kernel(inputs, emb_table, W):
    batch, ctx = inputs.shape
    vocab, embed = emb_table.shape
    info = plsc.get_sparse_core_info()
    nw = info.num_cores * info.num_subcores
    idx3 = inputs.astype(jnp.int32).reshape(
        nw, (batch // nw) * ctx // IDX_GRP, IDX_GRP)
    emb = _embed_sum(idx3, emb_table, batch, ctx, embed, nw, info.num_cores)
    return _project(emb, W)
